# Initial kernel scaffold; baseline (speedup 1.0000x reference)
import jax
import jax.numpy as jnp
from jax.experimental import pallas as pl


def kernel(x, enc_mask_token, W0, b0, g0, be0, a0, W1, b1, g1, be1, a1, Wd, bd, gd, bed, ad, edge_index):
    raise NotImplementedError("write your pallas kernel here")


# speedup vs baseline: 19.2052x; 19.2052x over previous
import functools

import numpy as np
import jax
import jax.numpy as jnp
from jax import lax
from jax.experimental import pallas as pl
from jax.experimental.pallas import tpu as pltpu
from jax.experimental.pallas import tpu_sc as plsc

_N = 10000
_E = 320000
_D = 128
_HID = 256
_NC = 2
_NS = 16
_NW = _NC * _NS
_NPAD = 10240
_CHUNK = 128
_NCHUNK = 79
_EPT = _NCHUNK * _CHUNK
_EPAD = _EPT * _NW
_NUM_MASK = 5000

_MASK_IDX = np.random.default_rng(42).permutation(_N)[:_NUM_MASK]
_MASKVEC = np.zeros((_N, 1), np.float32)
_MASKVEC[_MASK_IDX] = 1.0

_mesh = plsc.VectorSubcoreMesh(core_axis_name="c", subcore_axis_name="s",
                               num_cores=_NC, num_subcores=_NS)


@functools.partial(
    pl.kernel,
    out_type=jax.ShapeDtypeStruct((_NC, _NPAD, 16), jnp.float32),
    mesh=_mesh,
    scratch_types=[
        pltpu.VMEM((_NCHUNK, _CHUNK), jnp.int32),
        pltpu.VMEM((_CHUNK, 16), jnp.float32),
        pltpu.VMEM((_CHUNK, 16), jnp.float32),
        pltpu.VMEM_SHARED((_NPAD, 16), jnp.float32),
    ],
)
def _deg_sc(dst_hbm, out_hbm, dst_v, ones_v, zb_v, acc_sh):
    cid = lax.axis_index("c")
    sid = lax.axis_index("s")
    wid = cid * _NS + sid

    one16 = jnp.ones((16,), jnp.float32)
    zero16 = jnp.zeros((16,), jnp.float32)

    def fill_ones(i, carry):
        ones_v[i, :] = one16
        return carry

    lax.fori_loop(0, _CHUNK, fill_ones, 0)

    def fill_zero(i, carry):
        zb_v[i, :] = zero16
        return carry

    lax.fori_loop(0, _CHUNK, fill_zero, 0)

    def zinit(k, carry):
        pltpu.sync_copy(zb_v, acc_sh.at[pl.ds(sid * 640 + k * _CHUNK, _CHUNK)])
        return carry

    lax.fori_loop(0, 5, zinit, 0)
    plsc.subcore_barrier()

    pltpu.sync_copy(dst_hbm.at[wid], dst_v)

    def body(j, carry):
        pltpu.sync_copy(ones_v, acc_sh.at[dst_v.at[j]], add=True)
        return carry

    lax.fori_loop(0, _NCHUNK, body, 0)

    plsc.subcore_barrier()

    def out_body(k, carry):
        off = sid * 640 + k * _CHUNK
        pltpu.sync_copy(acc_sh.at[pl.ds(off, _CHUNK)], ones_v)
        pltpu.sync_copy(ones_v, out_hbm.at[cid, pl.ds(off, _CHUNK)])
        return carry

    lax.fori_loop(0, 5, out_body, 0)


@functools.partial(
    pl.kernel,
    out_type=jax.ShapeDtypeStruct((_NC, _NPAD, _D), jnp.float32),
    mesh=_mesh,
    scratch_types=[
        pltpu.VMEM((_NCHUNK, _CHUNK), jnp.int32),
        pltpu.VMEM((_NCHUNK, _CHUNK), jnp.int32),
        pltpu.VMEM((_CHUNK, _D), jnp.float32),
        pltpu.VMEM_SHARED((_NPAD, _D), jnp.float32),
        pltpu.SemaphoreType.DMA,
    ],
)
def _agg_sc(g_hbm, src_hbm, dst_hbm, out_hbm, src_v, dst_v, rows_v, acc_sh, sem):
    cid = lax.axis_index("c")
    sid = lax.axis_index("s")
    wid = cid * _NS + sid

    zero16 = jnp.zeros((16,), jnp.float32)

    def zbody(i, carry):
        r = i // 8
        c = (i % 8) * 16
        rows_v[r, pl.ds(c, 16)] = zero16
        return carry

    lax.fori_loop(0, _CHUNK * 8, zbody, 0)

    def zinit(k, carry):
        pltpu.sync_copy(rows_v, acc_sh.at[pl.ds(sid * 640 + k * _CHUNK, _CHUNK)])
        return carry

    lax.fori_loop(0, 5, zinit, 0)
    plsc.subcore_barrier()

    pltpu.sync_copy(src_hbm.at[wid], src_v)
    pltpu.sync_copy(dst_hbm.at[wid], dst_v)

    def body(j, carry):
        pltpu.async_copy(g_hbm.at[src_v.at[j]], rows_v, sem).wait()
        pltpu.sync_copy(rows_v, acc_sh.at[dst_v.at[j]], add=True)
        return carry

    lax.fori_loop(0, _NCHUNK, body, 0)

    plsc.subcore_barrier()

    def out_body(k, carry):
        off = sid * 640 + k * _CHUNK
        pltpu.sync_copy(acc_sh.at[pl.ds(off, _CHUNK)], rows_v)
        pltpu.sync_copy(rows_v, out_hbm.at[cid, pl.ds(off, _CHUNK)])
        return carry

    lax.fori_loop(0, 5, out_body, 0)


def _prep_body(x_ref, tok_ref, m_ref, c0_ref, c1_ref, dinv_ref, xm_ref, gv_ref):
    cnt = c0_ref[...] + c1_ref[...]
    dinv = lax.rsqrt(cnt + 1.0)
    m = m_ref[...]
    xm = x_ref[...] * (1.0 - m) + m * tok_ref[...]
    dinv_ref[...] = dinv
    xm_ref[...] = xm
    gv_ref[...] = xm * dinv


def _enc0_body(s0a, s0b, xm, dinv, w0, bias0, ga0, bt0, al0, w1, t1_ref, g1_ref):
    dv = dinv[...]
    u0 = dv * (s0a[...] + s0b[...]) + dv * dv * xm[...]
    t0 = jnp.dot(u0, w0[...], preferred_element_type=jnp.float32) + bias0[...]
    mu = jnp.mean(t0, axis=0, keepdims=True)
    d = t0 - mu
    var = jnp.mean(d * d, axis=0, keepdims=True)
    h = ga0[...] * d * lax.rsqrt(var + 1e-5) + bt0[...]
    h = jnp.where(h >= 0.0, h, al0[...] * h)
    t1 = jnp.dot(h, w1[...], preferred_element_type=jnp.float32)
    t1_ref[...] = t1
    g1_ref[...] = t1 * dv


def _enc1_body(s1a, s1b, t1, dinv, m_ref, bias1, ga1, bt1, al1, wd, t2_ref, g2_ref):
    dv = dinv[...]
    u1 = dv * (s1a[...] + s1b[...]) + dv * dv * t1[...] + bias1[...]
    mu = jnp.mean(u1, axis=0, keepdims=True)
    d = u1 - mu
    var = jnp.mean(d * d, axis=0, keepdims=True)
    h = ga1[...] * d * lax.rsqrt(var + 1e-5) + bt1[...]
    h = jnp.where(h >= 0.0, h, al1[...] * h)
    h = h * (1.0 - m_ref[...])
    t2 = jnp.dot(h, wd[...], preferred_element_type=jnp.float32)
    t2_ref[...] = t2
    g2_ref[...] = t2 * dv


def _dec_body(s2a, s2b, t2, dinv, m_ref, x_ref, biasd, gad, btd, ald, loss_ref):
    dv = dinv[...]
    u2 = dv * (s2a[...] + s2b[...]) + dv * dv * t2[...] + biasd[...]
    mu = jnp.mean(u2, axis=0, keepdims=True)
    d = u2 - mu
    var = jnp.mean(d * d, axis=0, keepdims=True)
    re = gad[...] * d * lax.rsqrt(var + 1e-5) + btd[...]
    re = jnp.where(re >= 0.0, re, ald[...] * re)
    x = x_ref[...]
    rn = jnp.maximum(jnp.sqrt(jnp.sum(re * re, axis=1, keepdims=True)), 1e-12)
    xn = jnp.maximum(jnp.sqrt(jnp.sum(x * x, axis=1, keepdims=True)), 1e-12)
    cos = jnp.sum((re / rn) * (x / xn), axis=1, keepdims=True)
    loss = jnp.sum(m_ref[...] * (1.0 - cos)) * (1.0 / _NUM_MASK)
    loss_ref[...] = jnp.reshape(loss, (1, 1))


def kernel(x, enc_mask_token, W0, b0, g0, be0, a0, W1, b1, g1, be1, a1,
           Wd, bd, gd, bed, ad, edge_index):
    f32 = jnp.float32
    src = edge_index[0].astype(jnp.int32)
    dst = edge_index[1].astype(jnp.int32)
    npad = _EPAD - _E
    pad_idx = _N + (jnp.arange(npad, dtype=jnp.int32) % 64)
    src_p = jnp.concatenate([src, pad_idx]).reshape(_NW, _NCHUNK, _CHUNK)
    dst_p = jnp.concatenate([dst, pad_idx]).reshape(_NW, _NCHUNK, _CHUNK)

    mvec = jnp.asarray(_MASKVEC)
    tok = enc_mask_token.reshape(1, _D)

    degout = _deg_sc(dst_p)
    c0 = degout[0, :_N, 0:1]
    c1 = degout[1, :_N, 0:1]

    dinv, xm, gv = pl.pallas_call(
        _prep_body,
        out_shape=[jax.ShapeDtypeStruct((_N, 1), f32),
                   jax.ShapeDtypeStruct((_N, _D), f32),
                   jax.ShapeDtypeStruct((_N, _D), f32)],
    )(x, tok, mvec, c0, c1)

    rowpad = ((0, _NPAD - _N), (0, 0))
    s0 = _agg_sc(jnp.pad(gv, rowpad), src_p, dst_p)

    t1, g1v = pl.pallas_call(
        _enc0_body,
        out_shape=[jax.ShapeDtypeStruct((_N, _D), f32),
                   jax.ShapeDtypeStruct((_N, _D), f32)],
    )(s0[0, :_N], s0[1, :_N], xm, dinv,
      W0, b0.reshape(1, _HID), g0.reshape(1, _HID), be0.reshape(1, _HID),
      jnp.reshape(a0, (1, 1)), W1)

    s1 = _agg_sc(jnp.pad(g1v, rowpad), src_p, dst_p)

    t2, g2v = pl.pallas_call(
        _enc1_body,
        out_shape=[jax.ShapeDtypeStruct((_N, _D), f32),
                   jax.ShapeDtypeStruct((_N, _D), f32)],
    )(s1[0, :_N], s1[1, :_N], t1, dinv, mvec,
      b1.reshape(1, _D), g1.reshape(1, _D), be1.reshape(1, _D),
      jnp.reshape(a1, (1, 1)), Wd)

    s2 = _agg_sc(jnp.pad(g2v, rowpad), src_p, dst_p)

    loss = pl.pallas_call(
        _dec_body,
        out_shape=jax.ShapeDtypeStruct((1, 1), f32),
    )(s2[0, :_N], s2[1, :_N], t2, dinv, mvec, x,
      bd.reshape(1, _D), gd.reshape(1, _D), bed.reshape(1, _D),
      jnp.reshape(ad, (1, 1)))
    return loss[0, 0]
